# baseline (speedup 1.0000x reference)
import jax
import jax.numpy as jnp
from jax import lax
from jax.experimental import pallas as pl
from jax.experimental.pallas import tpu as pltpu
from jax.experimental.pallas import tpu_sc as plsc

_SEQ = 1729
_PATCH = 576
_B = 32
_D = 768
_LANES = 16
_NVEC = _D // _LANES
_NW = 32
_NTOK = 55
_NBUF = 5


def _sc_body(xt_hbm, pe_hbm, out_hbm, pe_v, bufs, sins, souts):
    nc = 2
    wid = lax.axis_index("s") * nc + lax.axis_index("c")
    base = wid * (_NTOK - 1)

    pltpu.sync_copy(pe_hbm, pe_v)

    def in_start(c, b):
        pltpu.make_async_copy(
            xt_hbm.at[pl.ds(base + c, 1)], bufs[b], sins[b]
        ).start()

    def in_wait(b):
        pltpu.make_async_copy(
            xt_hbm.at[pl.ds(0, 1)], bufs[b], sins[b]
        ).wait()

    def out_start(c, b):
        pltpu.make_async_copy(
            bufs[b], out_hbm.at[pl.ds(base + c, 1)], souts[b]
        ).start()

    def out_wait(b):
        pltpu.make_async_copy(
            bufs[b], out_hbm.at[pl.ds(0, 1)], souts[b]
        ).wait()

    def compute(c, b):
        t = base + c
        pos = (t + _PATCH - 1) // _PATCH
        vals = [pe_v[pos, pl.ds(k * _LANES, _LANES)] for k in range(_NVEC)]
        buf = bufs[b]

        def body(j, carry):
            for k in range(_NVEC):
                buf[0, j, pl.ds(k * _LANES, _LANES)] += vals[k]
            return carry

        lax.fori_loop(0, _B, body, jnp.int32(0))

    def step(j, par, c2_valid, c2_wait):
        b2 = (par + 2) % _NBUF
        if c2_wait:
            out_wait(b2)
        if c2_valid:
            in_start(j + 2, b2)
        b = par % _NBUF
        in_wait(b)
        compute(j, b)
        out_start(j, b)

    in_start(0, 0)
    in_start(1, 1)

    def flags(j):
        c2 = j + 2
        return c2 < _NTOK, c2 < _NTOK and c2 - _NBUF >= 0

    head_end = _NBUF - 2
    for j in range(head_end):
        v, w = flags(j)
        step(j, j % _NBUF, v, w)

    main_last = _NTOK - 3
    ngroups = (main_last + 1 - head_end) // _NBUF
    main_end = head_end + ngroups * _NBUF

    def group(m, carry):
        j0 = head_end + _NBUF * m
        for u in range(_NBUF):
            step(j0 + u, (head_end + u) % _NBUF, True, True)
        return carry

    lax.fori_loop(0, ngroups, group, jnp.int32(0))

    for j in range(main_end, _NTOK):
        v, w = flags(j)
        step(j, j % _NBUF, v, w)

    outstanding = [0] * _NBUF
    for c in range(_NTOK):
        outstanding[c % _NBUF] += 1
    for j in range(_NTOK):
        v, w = flags(j)
        if w:
            outstanding[(j + 2) % _NBUF] -= 1
    for b in range(_NBUF):
        for _ in range(outstanding[b]):
            out_wait(b)


def kernel(x, pe):
    B, S, D = x.shape
    xt = jnp.transpose(x, (1, 0, 2))
    mesh = plsc.VectorSubcoreMesh(core_axis_name="c", subcore_axis_name="s")
    sc_add = pl.kernel(
        _sc_body,
        out_type=jax.ShapeDtypeStruct((S, B, D), x.dtype),
        mesh=mesh,
        scratch_types=[
            pltpu.VMEM((4, D), jnp.float32),
            [pltpu.VMEM((1, B, D), jnp.float32) for _ in range(_NBUF)],
            [pltpu.SemaphoreType.DMA for _ in range(_NBUF)],
            [pltpu.SemaphoreType.DMA for _ in range(_NBUF)],
        ],
    )
    out_t = sc_add(xt, pe)
    return jnp.transpose(out_t, (1, 0, 2))
